# baseline (speedup 1.0000x reference)
import functools

import jax
import jax.numpy as jnp
from jax import lax
from jax.experimental import pallas as pl
from jax.experimental.pallas import tpu as pltpu
from jax.experimental.pallas import tpu_sc as plsc

R = 1024
N = 32768
L = 16
NW = 32
ROWS_PER_W = R // NW
STEPS = N // L


def _vperm(x, perm):
    return lax.gather(
        x,
        perm[:, None],
        dimension_numbers=lax.GatherDimensionNumbers(
            offset_dims=(), collapsed_slice_dims=(0,), start_index_map=(0,)
        ),
        slice_sizes=(1,),
        mode=lax.GatherScatterMode.PROMISE_IN_BOUNDS,
    )


def _lane_argmax(cm, ci, iota):
    for off in (8, 4, 2, 1):
        perm = iota ^ off
        om = _vperm(cm, perm)
        oi = _vperm(ci, perm)
        take = (om > cm) | ((om == cm) & (oi < ci))
        cm = jnp.where(take, om, cm)
        ci = jnp.where(take, oi, ci)
    return cm, ci


def _sc_argmax(x_hbm, out_hbm, row_v, res_v):
    c = lax.axis_index("c")
    s = lax.axis_index("s")
    wid = s * 2 + c
    base = wid * ROWS_PER_W
    iota = lax.iota(jnp.int32, L)
    neg_inf = jnp.full((L,), -jnp.inf, dtype=jnp.float32)
    zeros_i = jnp.zeros((L,), dtype=jnp.int32)
    big = jnp.full((L,), jnp.int32(2**30), dtype=jnp.int32)

    def do_group(g, _):
        def do_row(r, resvec):
            row = base + g * L + r
            pltpu.sync_copy(x_hbm.at[row], row_v)

            def step(i, carry):
                cm, ci = carry
                v = row_v[pl.ds(i * L, L)]
                gt = v > cm
                cm = jnp.where(gt, v, cm)
                ci = jnp.where(gt, iota + i * L, ci)
                return cm, ci

            cm, ci = lax.fori_loop(0, STEPS, step, (neg_inf, zeros_i))
            _, p = _lane_argmax(cm, ci, iota)
            return jnp.where(iota == r, p, resvec)

        resvec = lax.fori_loop(0, L, do_row, zeros_i)
        res_v[pl.ds(g * L, L)] = resvec
        return 0

    lax.fori_loop(0, ROWS_PER_W // L, do_group, 0)
    pltpu.sync_copy(res_v, out_hbm.at[pl.ds(base, ROWS_PER_W)])


@jax.jit
def _argmax_rows(x2d):
    mesh = plsc.VectorSubcoreMesh(core_axis_name="c", subcore_axis_name="s")
    f = pl.kernel(
        _sc_argmax,
        out_type=jax.ShapeDtypeStruct((R,), jnp.int32),
        mesh=mesh,
        scratch_types=[
            pltpu.VMEM((N,), jnp.float32),
            pltpu.VMEM((ROWS_PER_W,), jnp.int32),
        ],
    )
    return f(x2d)


def kernel(x):
    idx = _argmax_rows(x.reshape(R, N))
    return idx.reshape(128, 8).astype(jnp.int64)


# speedup vs baseline: 4.5633x; 4.5633x over previous
import functools

import jax
import jax.numpy as jnp
from jax import lax
from jax.experimental import pallas as pl
from jax.experimental.pallas import tpu as pltpu
from jax.experimental.pallas import tpu_sc as plsc

R = 1024
N = 32768
L = 16
NW = 32
ROWS_PER_W = R // NW
UNROLL = 8
ITERS = N // (L * UNROLL)


def _vperm(x, perm):
    return lax.gather(
        x,
        perm[:, None],
        dimension_numbers=lax.GatherDimensionNumbers(
            offset_dims=(), collapsed_slice_dims=(0,), start_index_map=(0,)
        ),
        slice_sizes=(1,),
        mode=lax.GatherScatterMode.PROMISE_IN_BOUNDS,
    )


def _merge(am, ai, bm, bi):
    take = (bm > am) | ((bm == am) & (bi < ai))
    return jnp.where(take, bm, am), jnp.where(take, bi, ai)


def _row_argmax(buf, iota):
    neg_inf = jnp.full((L,), -jnp.inf, dtype=jnp.float32)
    zero_i = jnp.zeros((L,), dtype=jnp.int32)

    def step(t, carry):
        viter = carry[-1]
        cms = list(carry[0])
        cis = list(carry[1])
        for j in range(UNROLL):
            v = buf[pl.ds(t * (L * UNROLL) + j * L, L)]
            gt = v > cms[j]
            cms[j] = jnp.where(gt, v, cms[j])
            cis[j] = jnp.where(gt, viter, cis[j])
        return (tuple(cms), tuple(cis), viter + 1)

    init = (
        tuple([neg_inf] * UNROLL),
        tuple([zero_i] * UNROLL),
        zero_i,
    )
    cms, cis, _ = lax.fori_loop(0, ITERS, step, init)

    bm, bi = None, None
    for j in range(UNROLL):
        idx = (cis[j] << 7) | (j << 4) | iota
        if bm is None:
            bm, bi = cms[j], idx
        else:
            bm, bi = _merge(bm, bi, cms[j], idx)

    for off in (8, 4, 2, 1):
        perm = iota ^ off
        bm, bi = _merge(bm, bi, _vperm(bm, perm), _vperm(bi, perm))
    return bi


def _sc_argmax(x_hbm, out_hbm, buf_a, buf_b, res_v, sem_a, sem_b):
    c = lax.axis_index("c")
    s = lax.axis_index("s")
    wid = s * 2 + c
    base = wid * ROWS_PER_W
    iota = lax.iota(jnp.int32, L)

    bufs = (buf_a, buf_b)
    sems = (sem_a, sem_b)

    def start(r, b):
        return pltpu.async_copy(x_hbm.at[base + r], bufs[b], sems[b])

    handles = [start(0, 0), None]
    res = [jnp.zeros((L,), jnp.int32), jnp.zeros((L,), jnp.int32)]
    for r in range(ROWS_PER_W):
        b = r & 1
        if r + 1 < ROWS_PER_W:
            handles[1 - b] = start(r + 1, 1 - b)
        handles[b].wait()
        p = _row_argmax(bufs[b], iota)
        res[r // L] = jnp.where(iota == (r % L), p, res[r // L])

    res_v[pl.ds(0, L)] = res[0]
    res_v[pl.ds(L, L)] = res[1]
    pltpu.sync_copy(res_v, out_hbm.at[pl.ds(base, ROWS_PER_W)])


@jax.jit
def _argmax_rows(x2d):
    mesh = plsc.VectorSubcoreMesh(core_axis_name="c", subcore_axis_name="s")
    f = pl.kernel(
        _sc_argmax,
        out_type=jax.ShapeDtypeStruct((R,), jnp.int32),
        mesh=mesh,
        scratch_types=[
            pltpu.VMEM((N,), jnp.float32),
            pltpu.VMEM((N,), jnp.float32),
            pltpu.VMEM((ROWS_PER_W,), jnp.int32),
            pltpu.SemaphoreType.DMA,
            pltpu.SemaphoreType.DMA,
        ],
    )
    return f(x2d)


def kernel(x):
    idx = _argmax_rows(x.reshape(R, N))
    return idx.reshape(128, 8).astype(jnp.int64)
